# baseline (speedup 1.0000x reference)
import functools

import jax
import jax.numpy as jnp
from jax import lax
from jax.experimental import pallas as pl
from jax.experimental.pallas import tpu as pltpu
from jax.experimental.pallas import tpu_sc as plsc

_INFO = plsc.get_sparse_core_info()
_NC = _INFO.num_cores
_NS = _INFO.num_subcores
_NW = _NC * _NS
_L = _INFO.num_lanes
_NCHUNK = 4


def _make_lookup(batch, nodes, dim):
    b_per_w = batch // _NW
    rows_per_chunk = b_per_w // _NCHUNK
    blocks_per_chunk = rows_per_chunk // _L
    mesh = plsc.VectorSubcoreMesh(core_axis_name="c", subcore_axis_name="s")

    @functools.partial(
        pl.kernel,
        mesh=mesh,
        out_type=jax.ShapeDtypeStruct((batch, dim), jnp.float32),
        compiler_params=pltpu.CompilerParams(needs_layout_passes=False),
        scratch_types=[
            pltpu.VMEM((b_per_w,), jnp.int32),
            pltpu.VMEM((nodes, dim), jnp.float32),
            pltpu.VMEM((b_per_w, dim), jnp.float32),
            pltpu.SemaphoreType.DMA,
            pltpu.SemaphoreType.DMA,
        ],
    )
    def lookup(idx_hbm, table_hbm, out_hbm, idx_v, table_v, out_v, lsem, osem):
        wid = lax.axis_index("s") * _NC + lax.axis_index("c")
        base = wid * b_per_w
        ld_idx = pltpu.async_copy(idx_hbm.at[wid], idx_v, lsem)
        ld_tab = pltpu.async_copy(table_hbm, table_v, lsem)
        ld_idx.wait()
        ld_tab.wait()

        lanes = lax.iota(jnp.int32, _L)

        def block(i, _):
            idxv = idx_v[pl.ds(i * _L, _L)]
            rows = i * _L + lanes
            for c in range(dim):
                col = jnp.full((_L,), c, jnp.int32)
                vals = plsc.load_gather(table_v, [idxv, col])
                plsc.store_scatter(out_v, [rows, col], vals)
            return 0

        copies = []
        for h in range(_NCHUNK):
            lax.fori_loop(h * blocks_per_chunk, (h + 1) * blocks_per_chunk,
                          block, 0)
            row0 = h * rows_per_chunk
            copies.append(
                pltpu.async_copy(
                    out_v.at[pl.ds(row0, rows_per_chunk)],
                    out_hbm.at[pl.ds(base + row0, rows_per_chunk)],
                    osem,
                )
            )
        for cp in copies:
            cp.wait()

    return lookup


def kernel(elements, emb):
    batch = elements.shape[0]
    nodes, dim = emb.shape
    idx2d = elements.reshape(_NW, batch // _NW)
    return _make_lookup(batch, nodes, dim)(idx2d, emb)


# speedup vs baseline: 2.3598x; 2.3598x over previous
import functools

import jax
import jax.numpy as jnp
from jax import lax
from jax.experimental import pallas as pl
from jax.experimental.pallas import tpu as pltpu
from jax.experimental.pallas import tpu_sc as plsc

_INFO = plsc.get_sparse_core_info()
_NC = _INFO.num_cores
_NS = _INFO.num_subcores
_NW = _NC * _NS
_L = _INFO.num_lanes
_NCHUNK = 4


def _make_lookup(batch, nodes, dim):
    b_per_w = batch // _NW
    rows_per_chunk = b_per_w // _NCHUNK
    blocks_per_chunk = rows_per_chunk // _L
    mesh = plsc.VectorSubcoreMesh(core_axis_name="c", subcore_axis_name="s")

    @functools.partial(
        pl.kernel,
        mesh=mesh,
        out_type=jax.ShapeDtypeStruct((batch, dim), jnp.float32),
        compiler_params=pltpu.CompilerParams(needs_layout_passes=False),
        scratch_types=[
            pltpu.VMEM((b_per_w,), jnp.int32),
            pltpu.VMEM((nodes, dim), jnp.float32),
            pltpu.VMEM((b_per_w, dim), jnp.float32),
            pltpu.SemaphoreType.DMA,
            pltpu.SemaphoreType.DMA,
        ],
    )
    def lookup(idx_hbm, table_hbm, out_hbm, idx_v, table_v, out_v, lsem, osem):
        wid = lax.axis_index("s") * _NC + lax.axis_index("c")
        base = wid * b_per_w
        ld_idx = pltpu.async_copy(idx_hbm.at[wid], idx_v, lsem)
        ld_tab = pltpu.async_copy(table_hbm, table_v, lsem)
        ld_idx.wait()
        ld_tab.wait()

        lanes = lax.iota(jnp.int32, _L)

        def block(i, _):
            idxv = idx_v[pl.ds(i * _L, _L)]
            rows = i * _L + lanes
            for c in range(dim):
                col = (lanes + c) & (dim - 1)
                vals = plsc.load_gather(table_v, [idxv, col])
                plsc.store_scatter(out_v, [rows, col], vals)
            return 0

        copies = []
        for h in range(_NCHUNK):
            lax.fori_loop(h * blocks_per_chunk, (h + 1) * blocks_per_chunk,
                          block, 0)
            row0 = h * rows_per_chunk
            copies.append(
                pltpu.async_copy(
                    out_v.at[pl.ds(row0, rows_per_chunk)],
                    out_hbm.at[pl.ds(base + row0, rows_per_chunk)],
                    osem,
                )
            )
        for cp in copies:
            cp.wait()

    return lookup


def kernel(elements, emb):
    batch = elements.shape[0]
    nodes, dim = emb.shape
    idx2d = elements.reshape(_NW, batch // _NW)
    return _make_lookup(batch, nodes, dim)(idx2d, emb)
